# Initial kernel scaffold; baseline (speedup 1.0000x reference)
import jax
import jax.numpy as jnp
from jax.experimental import pallas as pl


def kernel(x, params, edge_index, edge_types, graph_ids):
    raise NotImplementedError("write your pallas kernel here")


# speedup vs baseline: 15.9533x; 15.9533x over previous
import functools

import jax
import jax.numpy as jnp
import numpy as np
from jax import lax
from jax.experimental import pallas as pl
from jax.experimental.pallas import tpu as pltpu
from jax.experimental.pallas import tpu_sc as plsc

N = 10000
E = 320000
D = 128
H = 10
DH = 12
T = 16
G = 64
FF = 2 * D
L = 16
DP = H * L
W_SAG = D + 16
WSA = 80
WSB = 64

NC = 2
NS = 16
NW = NC * NS
EW = E // NW
C = 400
NCH = EW // C

EB = 4000
NB = 2000
NNB = N // NB
NP = 10240
PB = 256
NPB = NP // PB

ISQ = float(1.0 / np.sqrt(DH))
NEG = -1e9

f32 = jnp.float32
i32 = jnp.int32
u32 = jnp.uint32

_PREC = lax.Precision.HIGHEST


def _dot(a, b):
    return jnp.dot(a, b, precision=_PREC, preferred_element_type=f32)


def _sel_m():
    S = np.zeros((DP, 16), np.float32)
    for h in range(H):
        S[h * L:(h + 1) * L, h] = 1.0
    S[15, 0] = 0.0
    S[15, 10] = 1.0
    return S


def _expand_m():
    M = np.zeros((16, DP), np.float32)
    for h in range(H):
        M[h, h * L:(h + 1) * L] = 1.0
    return M


_SELM_NP = _sel_m()
_EXPM_NP = _expand_m()


def _qkv_body(x_ref, nm_ref, wq_ref, wk_ref, wv_ref, q_ref, k_ref, v_ref):
    x = x_ref[...]
    nm = nm_ref[...]
    q = _dot(x, wq_ref[...])
    k = _dot(x, wk_ref[...])
    col = lax.broadcasted_iota(i32, (NB, DP), 1)
    q_ref[...] = jnp.where(col == 15, nm, q)
    k_ref[...] = jnp.where(col == 15, nm, k)
    v_ref[...] = _dot(x, wv_ref[...])


def _qkv_call(x, nm, wq, wk, wv):
    return pl.pallas_call(
        _qkv_body,
        grid=(NNB,),
        in_specs=[
            pl.BlockSpec((NB, D), lambda i: (i, 0)),
            pl.BlockSpec((NB, 1), lambda i: (i, 0)),
            pl.BlockSpec((D, DP), lambda i: (0, 0)),
            pl.BlockSpec((D, DP), lambda i: (0, 0)),
            pl.BlockSpec((D, DP), lambda i: (0, 0)),
        ],
        out_specs=[pl.BlockSpec((NB, DP), lambda i: (i, 0))] * 3,
        out_shape=[jax.ShapeDtypeStruct((N, DP), f32)] * 3,
    )(x, nm, wq, wk, wv)


def _ex_body(qd_ref, ks_ref, et_ref, sel_ref, eb_ref, ex_ref):
    P = qd_ref[...] * ks_ref[...]
    s16 = _dot(P, sel_ref[...])
    col16 = lax.broadcasted_iota(i32, (EB, 16), 1)
    oh = (et_ref[...] == col16).astype(f32)
    ebg = _dot(oh, eb_ref[...])
    em = s16[:, 10:11]
    exv = jnp.exp(s16 * ISQ + ebg) * em
    ex_ref[...] = jnp.where(col16 < 10, exv, 0.0)


def _ex_call(qd, ks, et2d, selm, ebpad):
    return pl.pallas_call(
        _ex_body,
        grid=(E // EB,),
        in_specs=[
            pl.BlockSpec((EB, DP), lambda i: (i, 0)),
            pl.BlockSpec((EB, DP), lambda i: (i, 0)),
            pl.BlockSpec((EB, 1), lambda i: (i, 0)),
            pl.BlockSpec((DP, 16), lambda i: (0, 0)),
            pl.BlockSpec((16, 16), lambda i: (0, 0)),
        ],
        out_specs=pl.BlockSpec((EB, 16), lambda i: (i, 0)),
        out_shape=jax.ShapeDtypeStruct((E, 16), f32),
    )(qd, ks, et2d, selm, ebpad)


def _msg_body(vs_ref, ex_ref, em_ref, ma_ref, mb_ref):
    m = vs_ref[...] * _dot(ex_ref[...], em_ref[...])
    ma_ref[...] = m[:, 0:DP // 2]
    mb_ref[...] = m[:, DP // 2:DP]


def _msg_call(vs, ex, expm):
    return pl.pallas_call(
        _msg_body,
        grid=(E // EB,),
        in_specs=[
            pl.BlockSpec((EB, DP), lambda i: (i, 0)),
            pl.BlockSpec((EB, 16), lambda i: (i, 0)),
            pl.BlockSpec((16, DP), lambda i: (0, 0)),
        ],
        out_specs=[pl.BlockSpec((EB, DP // 2), lambda i: (i, 0))] * 2,
        out_shape=[jax.ShapeDtypeStruct((E, DP // 2), f32)] * 2,
    )(vs, ex, expm)


def _bn(h, g, b):
    m = jnp.mean(h, axis=0, keepdims=True)
    v = jnp.mean((h - m) * (h - m), axis=0, keepdims=True)
    return (h - m) * lax.rsqrt(v + 1e-5) * g + b


def _acc_stats(h, s_ref, q_ref):
    @pl.when(pl.program_id(0) == 0)
    def _init():
        s_ref[...] = jnp.zeros((1, D), f32)
        q_ref[...] = jnp.zeros((1, D), f32)

    s_ref[...] += jnp.sum(h, axis=0, keepdims=True)
    q_ref[...] += jnp.sum(h * h, axis=0, keepdims=True)


def _bn_from(h, s, q, g, b):
    m = s * (1.0 / N)
    v = q * (1.0 / N) - m * m
    return (h - m) * lax.rsqrt(v + 1e-5) * g + b


def _dense1_body(x_ref, a0a_ref, a1a_ref, a0b_ref, a1b_ref, d0_ref, d1_ref,
                 ema_ref, emb_ref, woa_ref, wob_ref, h1_ref, s_ref, q_ref):
    den = d0_ref[...] + d1_ref[...] + 1e-9
    agga = (a0a_ref[...] + a1a_ref[...]) / _dot(den, ema_ref[...])
    aggb = (a0b_ref[...] + a1b_ref[...]) / _dot(den, emb_ref[...])
    h1 = x_ref[...] + _dot(agga, woa_ref[...]) + _dot(aggb, wob_ref[...])
    h1_ref[...] = h1
    _acc_stats(h1, s_ref, q_ref)


def _dense2_body(h1_ref, s1_ref, q1_ref, g1_ref, b1_ref, f1_ref, fb1_ref,
                 f2_ref, fb2_ref, hp_ref, s_ref, q_ref):
    hn = _bn_from(h1_ref[...], s1_ref[...], q1_ref[...], g1_ref[...], b1_ref[...])
    h2 = _dot(jnp.maximum(_dot(hn, f1_ref[...]) + fb1_ref[...], 0.0),
              f2_ref[...]) + fb2_ref[...]
    hp = hn + h2
    hp_ref[...] = hp
    _acc_stats(hp, s_ref, q_ref)


def _dense3_body(hp_ref, s2_ref, q2_ref, g2_ref, b2_ref, nm_ref,
                 h_ref, saga_ref, sagb_ref):
    h = _bn_from(hp_ref[...], s2_ref[...], q2_ref[...], g2_ref[...], b2_ref[...])
    h_ref[...] = h
    nm = nm_ref[...]
    hm = h * nm
    saga_ref[...] = hm[:, 0:WSA]
    sagb_ref[:, 0:D - WSA] = hm[:, WSA:D]
    col = lax.broadcasted_iota(i32, (NB, WSB - (D - WSA)), 1)
    sagb_ref[:, D - WSA:WSB] = jnp.where(col == 0, nm, 0.0)


def _row(shape):
    return pl.BlockSpec((NB, shape), lambda i: (i, 0))


def _full(r, c):
    return pl.BlockSpec((r, c), lambda i: (0, 0))


_STAT = pl.BlockSpec((1, D), lambda i: (0, 0))


def _dense_call(x, nm, a0a, a1a, a0b, a1b, d0, d1, ema, emb, woa, wob,
                g1, b1, f1, fb1, f2, fb2, g2, b2):
    h1, s1, q1 = pl.pallas_call(
        _dense1_body,
        grid=(NNB,),
        in_specs=[_row(D), _row(DP // 2), _row(DP // 2), _row(DP // 2),
                  _row(DP // 2), _row(16), _row(16), _full(16, DP // 2),
                  _full(16, DP // 2), _full(DP // 2, D), _full(DP // 2, D)],
        out_specs=[_row(D), _STAT, _STAT],
        out_shape=[jax.ShapeDtypeStruct((N, D), f32),
                   jax.ShapeDtypeStruct((1, D), f32),
                   jax.ShapeDtypeStruct((1, D), f32)],
    )(x, a0a, a1a, a0b, a1b, d0, d1, ema, emb, woa, wob)
    hp, s2, q2 = pl.pallas_call(
        _dense2_body,
        grid=(NNB,),
        in_specs=[_row(D), _STAT, _STAT, _full(1, D), _full(1, D),
                  _full(D, FF), _full(1, FF), _full(FF, D), _full(1, D)],
        out_specs=[_row(D), _STAT, _STAT],
        out_shape=[jax.ShapeDtypeStruct((N, D), f32),
                   jax.ShapeDtypeStruct((1, D), f32),
                   jax.ShapeDtypeStruct((1, D), f32)],
    )(h1, s1, q1, g1, b1, f1, fb1, f2, fb2)
    return pl.pallas_call(
        _dense3_body,
        grid=(NNB,),
        in_specs=[_row(D), _STAT, _STAT, _full(1, D), _full(1, D), _row(1)],
        out_specs=[_row(D), _row(WSA), _row(WSB)],
        out_shape=[jax.ShapeDtypeStruct((N, D), f32),
                   jax.ShapeDtypeStruct((N, WSA), f32),
                   jax.ShapeDtypeStruct((N, WSB), f32)],
    )(hp, s2, q2, g2, b2, nm)


def _score_body(k_sel, h_ref, nm_ref, s0a_ref, s1a_ref, s0b_ref, s1b_ref,
                w_ref, b_ref, hn_ref, nmn_ref):
    saga = s0a_ref[...] + s1a_ref[...]
    sagb = s0b_ref[...] + s1b_ref[...]
    nm = nm_ref[...]
    deg = nm * sagb[:, D - WSA:D - WSA + 1]
    inv = nm / jnp.maximum(deg, 1.0)
    w = w_ref[...]
    sc = jnp.sum(saga * w[:, 0:WSA], axis=1, keepdims=True)
    sc += jnp.sum(sagb[:, 0:D - WSA] * w[:, WSA:D], axis=1, keepdims=True)
    score = sc * inv + b_ref[0, 0]
    score = jnp.where(nm > 0, score, NEG)

    uu = lax.bitcast_convert_type(score, u32)
    isneg = uu >= u32(0x80000000)
    ku = jnp.where(isneg, u32(0xFFFFFFFF) ^ uu, uu | u32(0x80000000))

    def bit_step(i, tv):
        cand = tv | (u32(1) << (u32(31) - u32(i)))
        cnt = jnp.sum((ku >= cand).astype(f32))
        return jnp.where(cnt >= k_sel, cand, tv)

    tv = lax.fori_loop(0, 32, bit_step, u32(0))
    eq = ku == tv
    n_eq = jnp.sum(eq.astype(f32))
    n_ge = jnp.sum((ku >= tv).astype(f32))
    r = k_sel - (n_ge - n_eq)
    idxv = lax.broadcasted_iota(i32, (N, 1), 0)

    def bit_step2(i, mv):
        cand = mv | (i32(1) << (i32(13) - i32(i)))
        cnt = jnp.sum((eq & (idxv < cand)).astype(f32))
        return jnp.where(cnt < r, cand, mv)

    mv = lax.fori_loop(0, 14, bit_step2, i32(0))
    newmask = ((ku > tv) | (eq & (idxv <= mv))).astype(f32) * nm
    hn_ref[...] = h_ref[...] * jnp.tanh(score) * newmask
    nmn_ref[...] = newmask


def _score_call(h, nm, s0a, s1a, s0b, s1b, w, b, k_sel):
    return pl.pallas_call(
        functools.partial(_score_body, k_sel),
        out_shape=[jax.ShapeDtypeStruct((N, D), f32),
                   jax.ShapeDtypeStruct((N, 1), f32)],
    )(h, nm, s0a, s1a, s0b, s1b, w, b)


def _pool_body(x_ref, gid_ref, nm_ref, sum_ref, max_ref, cnt_ref):
    @pl.when(pl.program_id(0) == 0)
    def _init():
        sum_ref[...] = jnp.zeros((G, D), f32)
        max_ref[...] = jnp.full((G, D), NEG, f32)
        cnt_ref[...] = jnp.zeros((G, D), f32)

    x = x_ref[...]
    gid = gid_ref[...]
    alive = nm_ref[...] > 0
    iotac = lax.broadcasted_iota(i32, (PB, G), 1)
    oht = ((gid == iotac) & alive).astype(f32)
    dn = (((0,), (0,)), ((), ()))
    sum_ref[...] += lax.dot_general(oht, x, dn, precision=_PREC,
                                    preferred_element_type=f32)
    cnt_ref[...] += lax.dot_general(oht, jnp.ones((PB, D), f32), dn,
                                    precision=_PREC, preferred_element_type=f32)

    def body(g, _):
        sel = (gid == g) & alive
        xm = jnp.max(jnp.where(sel, x, NEG), axis=0, keepdims=True)
        cur = max_ref[pl.ds(g, 1), :]
        max_ref[pl.ds(g, 1), :] = jnp.maximum(cur, xm)
        return 0

    lax.fori_loop(0, G, body, 0)


def _pool_call(xp, gidp, nmp):
    return pl.pallas_call(
        _pool_body,
        grid=(NPB,),
        in_specs=[
            pl.BlockSpec((PB, D), lambda i: (i, 0)),
            pl.BlockSpec((PB, 1), lambda i: (i, 0)),
            pl.BlockSpec((PB, 1), lambda i: (i, 0)),
        ],
        out_specs=[pl.BlockSpec((G, D), lambda i: (0, 0))] * 3,
        out_shape=[jax.ShapeDtypeStruct((G, D), f32)] * 3,
    )(xp, gidp, nmp)


def _head_body(s0, m0r, c0, s1, m1r, c1, s2, m2r, c2,
               M0a, M0b, m0b, M1r, m1b, M2r, m2b, o_ref):
    A = jnp.zeros((G, D), f32)
    B = jnp.zeros((G, D), f32)
    for s_ref, m_ref, c_ref in ((s0, m0r, c0), (s1, m1r, c1), (s2, m2r, c2)):
        cnt = c_ref[...][:, 0:1]
        A += s_ref[...] / jnp.maximum(cnt, 1.0)
        B += jnp.where(cnt > 0, m_ref[...], 0.0)
    A = jnp.maximum(A, 0.0)
    B = jnp.maximum(B, 0.0)
    o = jnp.maximum(_dot(A, M0a[...]) + _dot(B, M0b[...]) + m0b[...], 0.0)
    o = jnp.maximum(_dot(o, M1r[...]) + m1b[...], 0.0)
    o = _dot(o, M2r[...]) + m2b[...]
    mx = jnp.max(o, axis=1, keepdims=True)
    eo = jnp.exp(o - mx)
    o_ref[...] = eo / jnp.sum(eo, axis=1, keepdims=True)


def _head_call(pools, M0a, M0b, m0, M1, m1, M2, m2):
    flat = [a for triple in pools for a in triple]
    return pl.pallas_call(
        _head_body,
        out_shape=jax.ShapeDtypeStruct((G, 2), f32),
    )(*flat, M0a, M0b, m0, M1, m1, M2, m2)


_MESH = plsc.VectorSubcoreMesh(core_axis_name="c", subcore_axis_name="s")
_SC_PARAMS = pltpu.CompilerParams(use_tc_tiling_on_sc=False)
_ZROWS = N // NS


def _worker_id():
    return lax.axis_index("s") * NC + lax.axis_index("c")


def _sc_gather_qk_body(q_hbm, k_hbm, dst_hbm, src_hbm, qd_hbm, ks_hbm,
                       idx_v, rows_v, sem):
    wid = _worker_id()
    for i in range(NCH):
        base = pl.multiple_of(wid * EW + i * C, 8)
        pltpu.sync_copy(dst_hbm.at[pl.ds(base, C)], idx_v)
        pltpu.async_copy(q_hbm.at[idx_v], rows_v, sem).wait()
        pltpu.sync_copy(rows_v, qd_hbm.at[pl.ds(base, C)])
        pltpu.sync_copy(src_hbm.at[pl.ds(base, C)], idx_v)
        pltpu.async_copy(k_hbm.at[idx_v], rows_v, sem).wait()
        pltpu.sync_copy(rows_v, ks_hbm.at[pl.ds(base, C)])


_sc_gather_qk = pl.kernel(
    _sc_gather_qk_body,
    out_type=[jax.ShapeDtypeStruct((E, DP), f32)] * 2,
    mesh=_MESH,
    compiler_params=_SC_PARAMS,
    scratch_types=[pltpu.VMEM((C,), i32), pltpu.VMEM((C, DP), f32),
                   pltpu.SemaphoreType.DMA],
)


def _sc_den_vs_body(ex_hbm, v_hbm, dst_hbm, src_hbm, z16_hbm,
                    den0_hbm, den1_hbm, vs_hbm,
                    idx_v, exbuf, rows_v, sem, den_sp):
    c = lax.axis_index("c")
    s = lax.axis_index("s")
    wid = s * NC + c
    r0 = pl.multiple_of(s * _ZROWS, 1)
    pltpu.sync_copy(z16_hbm.at[pl.ds(r0, _ZROWS)], den_sp.at[pl.ds(r0, _ZROWS)])
    plsc.subcore_barrier()
    for i in range(NCH):
        base = pl.multiple_of(wid * EW + i * C, 8)
        pltpu.sync_copy(dst_hbm.at[pl.ds(base, C)], idx_v)
        pltpu.sync_copy(ex_hbm.at[pl.ds(base, C)], exbuf)
        pltpu.sync_copy(exbuf, den_sp.at[idx_v], add=True)
        pltpu.sync_copy(src_hbm.at[pl.ds(base, C)], idx_v)
        pltpu.async_copy(v_hbm.at[idx_v], rows_v, sem).wait()
        pltpu.sync_copy(rows_v, vs_hbm.at[pl.ds(base, C)])
    plsc.subcore_barrier()

    @pl.when(c == 0)
    def _out0():
        pltpu.sync_copy(den_sp.at[pl.ds(r0, _ZROWS)], den0_hbm.at[pl.ds(r0, _ZROWS)])

    @pl.when(c == 1)
    def _out1():
        pltpu.sync_copy(den_sp.at[pl.ds(r0, _ZROWS)], den1_hbm.at[pl.ds(r0, _ZROWS)])


_sc_den_vs = pl.kernel(
    _sc_den_vs_body,
    out_type=[jax.ShapeDtypeStruct((N, 16), f32),
              jax.ShapeDtypeStruct((N, 16), f32),
              jax.ShapeDtypeStruct((E, DP), f32)],
    mesh=_MESH,
    compiler_params=_SC_PARAMS,
    scratch_types=[pltpu.VMEM((C,), i32), pltpu.VMEM((C, 16), f32),
                   pltpu.VMEM((C, DP), f32), pltpu.SemaphoreType.DMA,
                   pltpu.VMEM_SHARED((N, 16), f32)],
)


def _sc_scatter_body(width, rows_hbm, dst_hbm, z_hbm, p0_hbm, p1_hbm,
                     idx_v, buf, acc_sp):
    c = lax.axis_index("c")
    s = lax.axis_index("s")
    wid = s * NC + c
    r0 = s * _ZROWS
    pltpu.sync_copy(z_hbm.at[pl.ds(r0, _ZROWS)], acc_sp.at[pl.ds(r0, _ZROWS)])
    plsc.subcore_barrier()
    for i in range(NCH):
        base = pl.multiple_of(wid * EW + i * C, 8)
        pltpu.sync_copy(dst_hbm.at[pl.ds(base, C)], idx_v)
        pltpu.sync_copy(rows_hbm.at[pl.ds(base, C)], buf)
        pltpu.sync_copy(buf, acc_sp.at[idx_v], add=True)
    plsc.subcore_barrier()

    @pl.when(c == 0)
    def _out0():
        pltpu.sync_copy(acc_sp.at[pl.ds(r0, _ZROWS)], p0_hbm.at[pl.ds(r0, _ZROWS)])

    @pl.when(c == 1)
    def _out1():
        pltpu.sync_copy(acc_sp.at[pl.ds(r0, _ZROWS)], p1_hbm.at[pl.ds(r0, _ZROWS)])


def _make_sc_scatter(width):
    return pl.kernel(
        functools.partial(_sc_scatter_body, width),
        out_type=[jax.ShapeDtypeStruct((N, width), f32)] * 2,
        mesh=_MESH,
        compiler_params=_SC_PARAMS,
        scratch_types=[pltpu.VMEM((C,), i32), pltpu.VMEM((C, width), f32),
                       pltpu.VMEM_SHARED((N, width), f32)],
    )


_sc_scatter_half = _make_sc_scatter(DP // 2)


def _sc_gs_body(width, tab_hbm, src_hbm, dst_hbm, z_hbm, p0_hbm, p1_hbm,
                idxs_v, idxd_v, buf, sem, acc_sp):
    c = lax.axis_index("c")
    s = lax.axis_index("s")
    wid = s * NC + c
    r0 = s * _ZROWS
    pltpu.sync_copy(z_hbm.at[pl.ds(r0, _ZROWS)], acc_sp.at[pl.ds(r0, _ZROWS)])
    plsc.subcore_barrier()
    for i in range(NCH):
        base = pl.multiple_of(wid * EW + i * C, 8)
        pltpu.sync_copy(src_hbm.at[pl.ds(base, C)], idxs_v)
        pltpu.async_copy(tab_hbm.at[idxs_v], buf, sem).wait()
        pltpu.sync_copy(dst_hbm.at[pl.ds(base, C)], idxd_v)
        pltpu.sync_copy(buf, acc_sp.at[idxd_v], add=True)
    plsc.subcore_barrier()

    @pl.when(c == 0)
    def _out0():
        pltpu.sync_copy(acc_sp.at[pl.ds(r0, _ZROWS)], p0_hbm.at[pl.ds(r0, _ZROWS)])

    @pl.when(c == 1)
    def _out1():
        pltpu.sync_copy(acc_sp.at[pl.ds(r0, _ZROWS)], p1_hbm.at[pl.ds(r0, _ZROWS)])


def _make_sc_gs(width):
    return pl.kernel(
        functools.partial(_sc_gs_body, width),
        out_type=[jax.ShapeDtypeStruct((N, width), f32)] * 2,
        mesh=_MESH,
        compiler_params=_SC_PARAMS,
        scratch_types=[pltpu.VMEM((C,), i32), pltpu.VMEM((C,), i32),
                       pltpu.VMEM((C, width), f32), pltpu.SemaphoreType.DMA,
                       pltpu.VMEM_SHARED((N, width), f32)],
    )


_sc_gs_a = _make_sc_gs(WSA)
_sc_gs_b = _make_sc_gs(WSB)


def _pad_w(Wx):
    return jnp.pad(Wx.reshape(D, H, DH), ((0, 0), (0, 0), (0, L - DH))).reshape(D, DP)


def _pad_wo(Wo):
    return jnp.pad(Wo.reshape(H, DH, D), ((0, 0), (0, L - DH), (0, 0))).reshape(DP, D)


def kernel(x, params, edge_index, edge_types, graph_ids):
    p = params
    src = edge_index[0]
    dst = edge_index[1]
    et2d = edge_types.reshape(E, 1).astype(i32)
    gidp = jnp.pad(graph_ids.astype(i32), (0, NP - N)).reshape(NP, 1)
    z16 = jnp.zeros((N, 16), f32)
    z80 = jnp.zeros((N, DP // 2), f32)
    zsa = jnp.zeros((N, WSA), f32)
    zsb = jnp.zeros((N, WSB), f32)
    selm = jnp.asarray(_SELM_NP)
    expm = jnp.asarray(_EXPM_NP)

    nmask = jnp.ones((N, 1), f32)
    ks_list = [N // 2, N // 4, N // 8]
    h = x
    pools = []
    for l in range(3):
        wq = _pad_w(p['Wq%d' % l])
        wk = _pad_w(p['Wk%d' % l])
        wv = _pad_w(p['Wv%d' % l])
        wo = _pad_wo(p['Wo%d' % l])
        ebpad = jnp.pad(p['Eb%d' % l], ((0, 0), (0, 16 - H)))
        q, k, v = _qkv_call(h, nmask, wq, wk, wv)
        qd, ks = _sc_gather_qk(q, k, dst, src)
        ex = _ex_call(qd, ks, et2d, selm, ebpad)
        den0, den1, vs = _sc_den_vs(ex, v, dst, src, z16)
        msga, msgb = _msg_call(vs, ex, expm)
        a0a, a1a = _sc_scatter_half(msga, dst, z80)
        a0b, a1b = _sc_scatter_half(msgb, dst, z80)
        h, saga_tab, sagb_tab = _dense_call(
            h, nmask, a0a, a1a, a0b, a1b, den0, den1,
            expm[:, 0:DP // 2], expm[:, DP // 2:DP],
            wo[0:DP // 2], wo[DP // 2:DP],
            p['g1_%d' % l].reshape(1, D), p['b1_%d' % l].reshape(1, D),
            p['F1_%d' % l], p['fb1_%d' % l].reshape(1, FF),
            p['F2_%d' % l], p['fb2_%d' % l].reshape(1, D),
            p['g2_%d' % l].reshape(1, D), p['b2_%d' % l].reshape(1, D))
        s0a, s1a = _sc_gs_a(saga_tab, src, dst, zsa)
        s0b, s1b = _sc_gs_b(sagb_tab, src, dst, zsb)
        h, nmask = _score_call(h, nmask, s0a, s1a, s0b, s1b,
                               p['pw%d' % l].reshape(1, D),
                               p['pb%d' % l].reshape(1, 1), ks_list[l])
        hp = jnp.pad(h, ((0, NP - N), (0, 0)))
        nmp = jnp.pad(nmask, ((0, NP - N), (0, 0)))
        pools.append(_pool_call(hp, gidp, nmp))

    return _head_call(pools, p['M0'][:D], p['M0'][D:],
                      p['m0'].reshape(1, D), p['M1'], p['m1'].reshape(1, D // 2),
                      p['M2'], p['m2'].reshape(1, 2))
